# baseline (speedup 1.0000x reference)
import functools

import jax
import jax.numpy as jnp
from jax import lax
from jax.experimental import pallas as pl
from jax.experimental.pallas import tpu as pltpu
from jax.experimental.pallas import tpu_sc as plsc

EMB_DIM = 64
CHUNK = 128
NBUF = 4
K = 2
BATCH = 4096
HIST = 200
BT = BATCH // CHUNK
N_CHUNKS_TOTAL = HIST * BT


@functools.cache
def _make_lookup():
    info = plsc.get_sparse_core_info()
    nc, ns = info.num_cores, info.num_subcores
    nw = nc * ns
    per_w = N_CHUNKS_TOTAL // nw
    assert per_w * nw == N_CHUNKS_TOTAL and per_w % NBUF == 0

    mesh = plsc.VectorSubcoreMesh(core_axis_name="c", subcore_axis_name="s")

    @functools.partial(
        pl.kernel,
        mesh=mesh,
        out_type=jax.ShapeDtypeStruct(
            (HIST, EMB_DIM // 8, BT, 8, CHUNK), jnp.float32
        ),
        scratch_types=[
            pltpu.VMEM((per_w, CHUNK), jnp.int32),
            pltpu.VMEM((NBUF, CHUNK, EMB_DIM), jnp.float32),
            pltpu.VMEM((NBUF, EMB_DIM // 8, 8, CHUNK), jnp.float32),
        ]
        + [pltpu.SemaphoreType.DMA] * (2 * NBUF),
        compiler_params=pltpu.CompilerParams(
            use_tc_tiling_on_sc=False, needs_layout_passes=False
        ),
    )
    def lookup(idx_hbm, table_hbm, out_hbm, idx_v, g_v, gt_v, *sems):
        gsem = sems[:NBUF]
        ssem = sems[NBUF:]
        wid = lax.axis_index("s") * nc + lax.axis_index("c")
        c0 = wid * per_w
        pltpu.sync_copy(idx_hbm.at[wid], idx_v)
        lanes = lax.iota(jnp.int32, 16)

        def gather_start(j, b):
            pltpu.async_copy(table_hbm.at[idx_v.at[j]], g_v.at[b], gsem[b])

        def gather_wait(j, b):
            pltpu.make_async_copy(
                table_hbm.at[idx_v.at[j]], g_v.at[b], gsem[b]
            ).wait()

        def transpose(b):
            def dt_body(dt, carry):
                for dr in range(8):
                    col = jnp.full((16,), dt * 8 + dr, jnp.int32)
                    for brg in range(CHUNK // 16):
                        rows = lanes + (brg * 16)
                        vals = plsc.load_gather(g_v.at[b], [rows, col])
                        gt_v[b, dt, dr, pl.ds(brg * 16, 16)] = vals
                return carry

            lax.fori_loop(0, EMB_DIM // 8, dt_body, 0, unroll=False)

        def hbt(j):
            c = c0 + j
            return c // BT, c % BT

        def scatter_start(j, b):
            h, bt = hbt(j)
            pltpu.async_copy(
                gt_v.at[b], out_hbm.at[h, :, bt], ssem[b]
            )

        def scatter_wait(j, b):
            h, bt = hbt(j)
            pltpu.make_async_copy(
                gt_v.at[b], out_hbm.at[h, :, bt], ssem[b]
            ).wait()

        def process(j, b):
            gather_wait(j, b)
            transpose(b)
            scatter_start(j, b)

        for j in range(NBUF):
            gather_start(j, j)
            if j >= K:
                process(j - K, j - K)

        n_blocks = (per_w - NBUF) // NBUF

        def blk_body(blk, carry):
            j0 = NBUF + blk * NBUF
            for b in range(NBUF):
                j = j0 + b
                scatter_wait(j - NBUF, b)
                gather_start(j, b)
                process(j - K, (b + NBUF - K) % NBUF)
            return carry

        lax.fori_loop(0, n_blocks, blk_body, 0, unroll=False)

        for j in range(per_w - K, per_w):
            process(j, j % NBUF)
        for b in range(NBUF):
            scatter_wait(per_w - NBUF + b, b)

    return lookup


def kernel(hidden_state, weight):
    info = plsc.get_sparse_core_info()
    nw = info.num_cores * info.num_subcores
    idx3 = hidden_state.astype(jnp.int32).T.reshape(nw, N_CHUNKS_TOTAL // nw, CHUNK)
    L = _make_lookup()(idx3, weight)
    return L.transpose(2, 4, 0, 1, 3).reshape(BATCH, HIST, EMB_DIM)


# speedup vs baseline: 1.5511x; 1.5511x over previous
import functools

import jax
import jax.numpy as jnp
from jax import lax
from jax.experimental import pallas as pl
from jax.experimental.pallas import tpu as pltpu
from jax.experimental.pallas import tpu_sc as plsc

EMB_DIM = 64
CHUNK = 128
NBUF = 4
K = 2
BATCH = 4096
HIST = 200
BT = BATCH // CHUNK
N_CHUNKS_TOTAL = HIST * BT


@functools.cache
def _make_lookup():
    info = plsc.get_sparse_core_info()
    nc, ns = info.num_cores, info.num_subcores
    nw = nc * ns
    per_w = N_CHUNKS_TOTAL // nw
    assert per_w * nw == N_CHUNKS_TOTAL and per_w % NBUF == 0

    mesh = plsc.VectorSubcoreMesh(core_axis_name="c", subcore_axis_name="s")

    @functools.partial(
        pl.kernel,
        mesh=mesh,
        out_type=jax.ShapeDtypeStruct(
            (HIST, EMB_DIM // 8, BT, 8, CHUNK), jnp.float32
        ),
        scratch_types=[
            pltpu.VMEM((per_w, CHUNK), jnp.int32),
            pltpu.VMEM((NBUF, CHUNK, EMB_DIM), jnp.float32),
            pltpu.VMEM((NBUF, EMB_DIM, CHUNK), jnp.float32),
        ]
        + [pltpu.SemaphoreType.DMA] * (2 * NBUF),
        compiler_params=pltpu.CompilerParams(
            use_tc_tiling_on_sc=False, needs_layout_passes=False
        ),
    )
    def lookup(idx_hbm, table_hbm, out_hbm, idx_v, g_v, gt_v, *sems):
        gsem = sems[:NBUF]
        ssem = sems[NBUF:]
        wid = lax.axis_index("s") * nc + lax.axis_index("c")
        c0 = wid * per_w
        pltpu.sync_copy(idx_hbm.at[wid], idx_v)
        lanes = lax.iota(jnp.int32, 16)

        def gather_start(j, b):
            pltpu.async_copy(table_hbm.at[idx_v.at[j]], g_v.at[b], gsem[b])

        def gather_wait(j, b):
            pltpu.make_async_copy(
                table_hbm.at[idx_v.at[j]], g_v.at[b], gsem[b]
            ).wait()

        def transpose(b):
            def tg_body(tg, carry):
                rows = lanes + tg * 16
                for dg in range(EMB_DIM // 16):
                    for k in range(16):
                        cols = (dg * 16) + ((lanes + k) & 15)
                        vals = plsc.load_gather(g_v.at[b], [rows, cols])
                        plsc.store_scatter(gt_v.at[b], [cols, rows], vals)
                return carry

            lax.fori_loop(0, CHUNK // 16, tg_body, 0, unroll=False)

        def hbt(j):
            c = c0 + j
            return c // BT, c % BT

        def scatter_start(j, b):
            h, bt = hbt(j)
            for dt in range(EMB_DIM // 8):
                pltpu.async_copy(
                    gt_v.at[b, pl.ds(dt * 8, 8)],
                    out_hbm.at[h, dt, bt],
                    ssem[b],
                )

        def scatter_wait(j, b):
            h, bt = hbt(j)
            for dt in range(EMB_DIM // 8):
                pltpu.make_async_copy(
                    gt_v.at[b, pl.ds(dt * 8, 8)],
                    out_hbm.at[h, dt, bt],
                    ssem[b],
                ).wait()

        def process(j, b):
            gather_wait(j, b)
            transpose(b)
            scatter_start(j, b)

        for j in range(NBUF):
            gather_start(j, j)
            if j >= K:
                process(j - K, j - K)

        n_blocks = (per_w - NBUF) // NBUF

        def blk_body(blk, carry):
            j0 = NBUF + blk * NBUF
            for b in range(NBUF):
                j = j0 + b
                scatter_wait(j - NBUF, b)
                gather_start(j, b)
                process(j - K, (b + NBUF - K) % NBUF)
            return carry

        lax.fori_loop(0, n_blocks, blk_body, 0, unroll=False)

        for j in range(per_w - K, per_w):
            process(j, j % NBUF)
        for b in range(NBUF):
            scatter_wait(per_w - NBUF + b, b)

    return lookup


def kernel(hidden_state, weight):
    info = plsc.get_sparse_core_info()
    nw = info.num_cores * info.num_subcores
    idx3 = hidden_state.astype(jnp.int32).T.reshape(nw, N_CHUNKS_TOTAL // nw, CHUNK)
    L = _make_lookup()(idx3, weight)
    return L.transpose(2, 4, 0, 1, 3).reshape(BATCH, HIST, EMB_DIM)


# speedup vs baseline: 1.7792x; 1.1470x over previous
import functools

import jax
import jax.numpy as jnp
from jax import lax
from jax.experimental import pallas as pl
from jax.experimental.pallas import tpu as pltpu
from jax.experimental.pallas import tpu_sc as plsc

EMB_DIM = 64
CHUNK = 128
NBUF = 4
K = 2
BATCH = 4096
HIST = 200
BT = BATCH // CHUNK
N_CHUNKS_TOTAL = HIST * BT


@functools.cache
def _make_lookup():
    info = plsc.get_sparse_core_info()
    nc, ns = info.num_cores, info.num_subcores
    nw = nc * ns
    per_w = N_CHUNKS_TOTAL // nw
    assert per_w * nw == N_CHUNKS_TOTAL and per_w % NBUF == 0

    mesh = plsc.VectorSubcoreMesh(core_axis_name="c", subcore_axis_name="s")

    @functools.partial(
        pl.kernel,
        mesh=mesh,
        out_type=jax.ShapeDtypeStruct(
            (HIST, EMB_DIM // 8, BT, 8 * CHUNK), jnp.float32
        ),
        scratch_types=[
            pltpu.VMEM((per_w, CHUNK), jnp.int32),
            pltpu.VMEM((NBUF, CHUNK, EMB_DIM), jnp.float32),
            pltpu.VMEM((NBUF, EMB_DIM * CHUNK), jnp.float32),
        ]
        + [pltpu.SemaphoreType.DMA] * (2 * NBUF),
        compiler_params=pltpu.CompilerParams(
            use_tc_tiling_on_sc=False, needs_layout_passes=False
        ),
    )
    def lookup(idx_hbm, table_hbm, out_hbm, idx_v, g_v, gt_v, *sems):
        gsem = sems[:NBUF]
        ssem = sems[NBUF:]
        wid = lax.axis_index("s") * nc + lax.axis_index("c")
        c0 = wid * per_w
        pltpu.sync_copy(idx_hbm.at[wid], idx_v)
        lanes = lax.iota(jnp.int32, 16)

        def gather_start(j, b):
            pltpu.async_copy(table_hbm.at[idx_v.at[j]], g_v.at[b], gsem[b])

        def gather_wait(j, b):
            pltpu.make_async_copy(
                table_hbm.at[idx_v.at[j]], g_v.at[b], gsem[b]
            ).wait()

        lanes128 = lanes * 128

        def transpose(b):
            def tg_body(tg, carry):
                tg16 = tg * 16
                for dg in range(EMB_DIM // 16):
                    cols = jnp.full((16,), dg * 16, jnp.int32) + lanes
                    stbase = lanes128 + (dg * 16 * CHUNK + tg16)
                    for k in range(16):
                        rot = (lanes + k) & 15
                        vals = plsc.load_gather(g_v.at[b], [rot + tg16, cols])
                        plsc.store_scatter(gt_v.at[b], [stbase + rot], vals)
                return carry

            lax.fori_loop(0, CHUNK // 16, tg_body, 0, unroll=False)

        def hbt(j):
            c = c0 + j
            return c // BT, c % BT

        def scatter_start(j, b):
            h, bt = hbt(j)
            for dt in range(EMB_DIM // 8):
                pltpu.async_copy(
                    gt_v.at[b, pl.ds(dt * 8 * CHUNK, 8 * CHUNK)],
                    out_hbm.at[h, dt, bt],
                    ssem[b],
                )

        def scatter_wait(j, b):
            h, bt = hbt(j)
            for dt in range(EMB_DIM // 8):
                pltpu.make_async_copy(
                    gt_v.at[b, pl.ds(dt * 8 * CHUNK, 8 * CHUNK)],
                    out_hbm.at[h, dt, bt],
                    ssem[b],
                ).wait()

        def process(j, b):
            gather_wait(j, b)
            transpose(b)
            scatter_start(j, b)

        for j in range(NBUF):
            gather_start(j, j)
            if j >= K:
                process(j - K, j - K)

        n_blocks = (per_w - NBUF) // NBUF

        def blk_body(blk, carry):
            j0 = NBUF + blk * NBUF
            for b in range(NBUF):
                j = j0 + b
                scatter_wait(j - NBUF, b)
                gather_start(j, b)
                process(j - K, (b + NBUF - K) % NBUF)
            return carry

        lax.fori_loop(0, n_blocks, blk_body, 0, unroll=False)

        for j in range(per_w - K, per_w):
            process(j, j % NBUF)
        for b in range(NBUF):
            scatter_wait(per_w - NBUF + b, b)

    return lookup


def kernel(hidden_state, weight):
    info = plsc.get_sparse_core_info()
    nw = info.num_cores * info.num_subcores
    idx3 = hidden_state.astype(jnp.int32).T.reshape(nw, N_CHUNKS_TOTAL // nw, CHUNK)
    L = _make_lookup()(idx3, weight)
    L5 = L.reshape(HIST, EMB_DIM // 8, BT, 8, CHUNK)
    return L5.transpose(2, 4, 0, 1, 3).reshape(BATCH, HIST, EMB_DIM)
